# Initial kernel scaffold; baseline (speedup 1.0000x reference)
import jax
import jax.numpy as jnp
from jax.experimental import pallas as pl


def kernel(responses, response_table, position_table):
    raise NotImplementedError("write your pallas kernel here")


# speedup vs baseline: 8.6868x; 8.6868x over previous
import functools

import jax
import jax.numpy as jnp
from jax import lax
from jax.experimental import pallas as pl
from jax.experimental.pallas import tpu as pltpu
from jax.experimental.pallas import tpu_sc as plsc

_B = 4096
_S = 200
_D = 128
_NC = 2
_NS = 16
_NW = _NC * _NS
_BW = _B // _NW
_L = 16
_DV = _D // _L


def _body(resp_t, table, pos, out, idx_v, pos_v, buf0, buf1, g0, g1, s0, s1):
    wid = lax.axis_index("s") * _NC + lax.axis_index("c")
    b0 = wid * _BW

    pltpu.sync_copy(resp_t.at[:, pl.ds(b0, _BW)], idx_v)
    pltpu.sync_copy(pos, pos_v)

    pltpu.async_copy(table.at[idx_v.at[0]], buf0, g0)
    pltpu.async_copy(table.at[idx_v.at[1]], buf1, g1)

    def step(s, buf, gsem, ssem):
        pltpu.make_async_copy(table.at[idx_v.at[s]], buf, gsem).wait()

        prow = [pos_v[s, pl.ds(_L * j, _L)] for j in range(_DV)]

        def add_row(r, carry):
            for j in range(_DV):
                sl = pl.ds(_L * j, _L)
                buf[r, sl] = buf[r, sl] + prow[j]
            return carry

        lax.fori_loop(0, _BW, add_row, 0, unroll=2)

        pltpu.async_copy(buf, out.at[pl.ds(b0, _BW), s], ssem)

        @pl.when(s < _S - 2)
        def _():
            pltpu.make_async_copy(buf, out.at[pl.ds(b0, _BW), s], ssem).wait()
            pltpu.async_copy(table.at[idx_v.at[s + 2]], buf, gsem)

    def loop_body(i, carry):
        step(2 * i, buf0, g0, s0)
        step(2 * i + 1, buf1, g1, s1)
        return carry

    lax.fori_loop(0, _S // 2, loop_body, 0)

    pltpu.make_async_copy(buf0, out.at[pl.ds(b0, _BW), _S - 2], s0).wait()
    pltpu.make_async_copy(buf1, out.at[pl.ds(b0, _BW), _S - 1], s1).wait()


@jax.jit
def _run(resp_t, table, pos):
    mesh = plsc.VectorSubcoreMesh(
        core_axis_name="c", subcore_axis_name="s",
        num_cores=_NC, num_subcores=_NS,
    )
    return pl.kernel(
        _body,
        out_type=jax.ShapeDtypeStruct((_B, _S, _D), jnp.float32),
        mesh=mesh,
        scratch_types=[
            pltpu.VMEM((_S, _BW), jnp.int32),
            pltpu.VMEM((_S, _D), jnp.float32),
            pltpu.VMEM((_BW, _D), jnp.float32),
            pltpu.VMEM((_BW, _D), jnp.float32),
            pltpu.SemaphoreType.DMA,
            pltpu.SemaphoreType.DMA,
            pltpu.SemaphoreType.DMA,
            pltpu.SemaphoreType.DMA,
        ],
    )(resp_t, table, pos)


def kernel(responses, response_table, position_table):
    resp_t = responses.astype(jnp.int32).T
    return _run(resp_t, response_table, position_table)


# speedup vs baseline: 9.3769x; 1.0794x over previous
import functools

import jax
import jax.numpy as jnp
from jax import lax
from jax.experimental import pallas as pl
from jax.experimental.pallas import tpu as pltpu
from jax.experimental.pallas import tpu_sc as plsc

_B = 4096
_S = 200
_D = 128
_NC = 2
_NS = 16
_NW = _NC * _NS
_BW = _B // _NW
_L = 16
_DV = _D // _L


_NBUF = 4


def _body(resp_t, table, pos, out, idx_v, pos_v, bufs, gsems, ssems):
    wid = lax.axis_index("s") * _NC + lax.axis_index("c")
    b0 = wid * _BW

    pltpu.sync_copy(resp_t.at[:, pl.ds(b0, _BW)], idx_v)
    pltpu.sync_copy(pos, pos_v)

    pltpu.async_copy(table.at[idx_v.at[0]], bufs[0], gsems[0])
    pltpu.async_copy(table.at[idx_v.at[1]], bufs[1], gsems[1])

    def step(s, p):
        buf = bufs[p]
        pltpu.make_async_copy(table.at[idx_v.at[s]], buf, gsems[p]).wait()

        prow = [pos_v[s, pl.ds(_L * j, _L)] for j in range(_DV)]

        def add_row(r, carry):
            for j in range(_DV):
                sl = pl.ds(_L * j, _L)
                buf[r, sl] = buf[r, sl] + prow[j]
            return carry

        lax.fori_loop(0, _BW, add_row, 0, unroll=4)

        pltpu.async_copy(buf, out.at[pl.ds(b0, _BW), s], ssems[p])

        q = (p + 2) % _NBUF
        bufq = bufs[q]

        @pl.when(s >= 2)
        def _():
            pltpu.make_async_copy(
                bufq, out.at[pl.ds(b0, _BW), s - 2], ssems[q]).wait()

        @pl.when(s < _S - 2)
        def _():
            pltpu.async_copy(table.at[idx_v.at[s + 2]], bufq, gsems[q])

    def loop_body(i, carry):
        for b in range(_NBUF):
            step(_NBUF * i + b, b)
        return carry

    lax.fori_loop(0, _S // _NBUF, loop_body, 0)

    pltpu.make_async_copy(
        bufs[2], out.at[pl.ds(b0, _BW), _S - 2], ssems[2]).wait()
    pltpu.make_async_copy(
        bufs[3], out.at[pl.ds(b0, _BW), _S - 1], ssems[3]).wait()


@jax.jit
def _run(resp_t, table, pos):
    mesh = plsc.VectorSubcoreMesh(
        core_axis_name="c", subcore_axis_name="s",
        num_cores=_NC, num_subcores=_NS,
    )
    return pl.kernel(
        _body,
        out_type=jax.ShapeDtypeStruct((_B, _S, _D), jnp.float32),
        mesh=mesh,
        scratch_types=[
            pltpu.VMEM((_S, _BW), jnp.int32),
            pltpu.VMEM((_S, _D), jnp.float32),
            tuple(pltpu.VMEM((_BW, _D), jnp.float32) for _ in range(_NBUF)),
            tuple(pltpu.SemaphoreType.DMA for _ in range(_NBUF)),
            tuple(pltpu.SemaphoreType.DMA for _ in range(_NBUF)),
        ],
    )(resp_t, table, pos)


def kernel(responses, response_table, position_table):
    resp_t = responses.astype(jnp.int32).T
    return _run(resp_t, response_table, position_table)


# speedup vs baseline: 9.3934x; 1.0018x over previous
import functools

import jax
import jax.numpy as jnp
from jax import lax
from jax.experimental import pallas as pl
from jax.experimental.pallas import tpu as pltpu
from jax.experimental.pallas import tpu_sc as plsc

_B = 4096
_S = 200
_D = 128
_NC = 2
_NS = 16
_NW = _NC * _NS
_BW = _B // _NW
_L = 16
_DV = _D // _L


_NBUF = 4


def _body(resp_t, table, pos, out, idx_v, pos_v, bufs, gsems, ssems):
    wid = lax.axis_index("s") * _NC + lax.axis_index("c")
    b0 = wid * _BW

    pltpu.sync_copy(resp_t.at[:, pl.ds(b0, _BW)], idx_v)
    pltpu.async_copy(table.at[idx_v.at[0]], bufs[0], gsems[0])
    pltpu.async_copy(table.at[idx_v.at[1]], bufs[1], gsems[1])
    pltpu.sync_copy(pos, pos_v)

    def step(s, p):
        buf = bufs[p]
        pltpu.make_async_copy(table.at[idx_v.at[s]], buf, gsems[p]).wait()

        prow = [pos_v[s, pl.ds(_L * j, _L)] for j in range(_DV)]

        def add_row(r, carry):
            for j in range(_DV):
                sl = pl.ds(_L * j, _L)
                buf[r, sl] = buf[r, sl] + prow[j]
            return carry

        lax.fori_loop(0, _BW, add_row, 0, unroll=4)

        pltpu.async_copy(buf, out.at[pl.ds(b0, _BW), s], ssems[p])

        q = (p + 2) % _NBUF
        bufq = bufs[q]

        @pl.when(s >= 2)
        def _():
            pltpu.make_async_copy(
                bufq, out.at[pl.ds(b0, _BW), s - 2], ssems[q]).wait()

        @pl.when(s < _S - 2)
        def _():
            pltpu.async_copy(table.at[idx_v.at[s + 2]], bufq, gsems[q])

    def loop_body(i, carry):
        for b in range(_NBUF):
            step(_NBUF * i + b, b)
        return carry

    lax.fori_loop(0, _S // _NBUF, loop_body, 0)

    pltpu.make_async_copy(
        bufs[2], out.at[pl.ds(b0, _BW), _S - 2], ssems[2]).wait()
    pltpu.make_async_copy(
        bufs[3], out.at[pl.ds(b0, _BW), _S - 1], ssems[3]).wait()


@jax.jit
def _run(resp_t, table, pos):
    mesh = plsc.VectorSubcoreMesh(
        core_axis_name="c", subcore_axis_name="s",
        num_cores=_NC, num_subcores=_NS,
    )
    return pl.kernel(
        _body,
        out_type=jax.ShapeDtypeStruct((_B, _S, _D), jnp.float32),
        mesh=mesh,
        scratch_types=[
            pltpu.VMEM((_S, _BW), jnp.int32),
            pltpu.VMEM((_S, _D), jnp.float32),
            tuple(pltpu.VMEM((_BW, _D), jnp.float32) for _ in range(_NBUF)),
            tuple(pltpu.SemaphoreType.DMA for _ in range(_NBUF)),
            tuple(pltpu.SemaphoreType.DMA for _ in range(_NBUF)),
        ],
    )(resp_t, table, pos)


def kernel(responses, response_table, position_table):
    resp_t = responses.astype(jnp.int32).T
    return _run(resp_t, response_table, position_table)


# speedup vs baseline: 9.4086x; 1.0016x over previous
import jax
import jax.numpy as jnp
from jax import lax
from jax.experimental import pallas as pl
from jax.experimental.pallas import tpu as pltpu
from jax.experimental.pallas import tpu_sc as plsc

_B = 4096
_S = 200
_D = 128
_NC = 2
_NS = 16
_NW = _NC * _NS
_BW = _B // _NW
_L = 16
_DV = _D // _L


_NBUF = 4


def _body(resp_t, table, pos, out, idx_v, pos_v, bufs, gsems, ssems):
    wid = lax.axis_index("s") * _NC + lax.axis_index("c")
    b0 = wid * _BW

    pltpu.sync_copy(resp_t.at[:, pl.ds(b0, _BW)], idx_v)
    pltpu.async_copy(table.at[idx_v.at[0]], bufs[0], gsems[0])
    pltpu.async_copy(table.at[idx_v.at[1]], bufs[1], gsems[1])
    pltpu.sync_copy(pos, pos_v)

    def step(s, p):
        buf = bufs[p]
        pltpu.make_async_copy(table.at[idx_v.at[s]], buf, gsems[p]).wait()

        prow = [pos_v[s, pl.ds(_L * j, _L)] for j in range(_DV)]

        def add_row(r, carry):
            for j in range(_DV):
                sl = pl.ds(_L * j, _L)
                buf[r, sl] = buf[r, sl] + prow[j]
            return carry

        lax.fori_loop(0, _BW, add_row, 0, unroll=4)

        pltpu.async_copy(buf, out.at[pl.ds(b0, _BW), s], ssems[p])

        q = (p + 2) % _NBUF
        bufq = bufs[q]

        @pl.when(s >= 2)
        def _():
            pltpu.make_async_copy(
                bufq, out.at[pl.ds(b0, _BW), s - 2], ssems[q]).wait()

        @pl.when(s < _S - 2)
        def _():
            pltpu.async_copy(table.at[idx_v.at[s + 2]], bufq, gsems[q])

    def loop_body(i, carry):
        for b in range(_NBUF):
            step(_NBUF * i + b, b)
        return carry

    lax.fori_loop(0, _S // _NBUF, loop_body, 0)

    pltpu.make_async_copy(
        bufs[2], out.at[pl.ds(b0, _BW), _S - 2], ssems[2]).wait()
    pltpu.make_async_copy(
        bufs[3], out.at[pl.ds(b0, _BW), _S - 1], ssems[3]).wait()


@jax.jit
def _run(resp_t, table, pos):
    mesh = plsc.VectorSubcoreMesh(
        core_axis_name="c", subcore_axis_name="s",
        num_cores=_NC, num_subcores=_NS,
    )
    return pl.kernel(
        _body,
        out_type=jax.ShapeDtypeStruct((_B, _S, _D), jnp.float32),
        mesh=mesh,
        scratch_types=[
            pltpu.VMEM((_S, _BW), jnp.int32),
            pltpu.VMEM((_S, _D), jnp.float32),
            tuple(pltpu.VMEM((_BW, _D), jnp.float32) for _ in range(_NBUF)),
            tuple(pltpu.SemaphoreType.DMA for _ in range(_NBUF)),
            tuple(pltpu.SemaphoreType.DMA for _ in range(_NBUF)),
        ],
    )(resp_t, table, pos)


def kernel(responses, response_table, position_table):
    resp_t = responses.astype(jnp.int32).T
    return _run(resp_t, response_table, position_table)
